# baseline (speedup 1.0000x reference)
import jax
import jax.numpy as jnp
from jax import lax
from jax.experimental import pallas as pl
from jax.experimental.pallas import tpu as pltpu
from jax.experimental.pallas import tpu_sc as plsc

_V = 100
_B = 16384
_NUM_CORES = 1
_NUM_SUBCORES = 16
_NUM_WORKERS = _NUM_CORES * _NUM_SUBCORES
_B_PER_W = _B // _NUM_WORKERS
_LANES = 16


def _lookup_body(labels_hbm, vocab_hbm, out_hbm, labels_v, table_v, out_v,
                 sem0):
    wid = lax.axis_index("s") * _NUM_CORES + lax.axis_index("c")
    base = wid * _B_PER_W
    cp_labels = pltpu.async_copy(labels_hbm.at[pl.ds(base, _B_PER_W)],
                                 labels_v, sem0)
    cp_table = pltpu.async_copy(vocab_hbm, table_v, sem0)
    cp_labels.wait()
    cp_table.wait()
    half = _B_PER_W // 2

    def gather_chunk(i, _):
        idx = labels_v[pl.ds(i * _LANES, _LANES)]
        out_v[pl.ds(i * _LANES, _LANES)] = plsc.load_gather(table_v, [idx])
        return 0

    out_cps = []
    for h, sem in ((0, sem0), (1, sem0)):
        lax.fori_loop(h * half // _LANES, (h + 1) * half // _LANES,
                      gather_chunk, 0, unroll=4)
        out_cps.append(pltpu.async_copy(
            out_v.at[pl.ds(h * half, half)],
            out_hbm.at[pl.ds(base + h * half, half)], sem))
    for cp in out_cps:
        cp.wait()


@jax.jit
def kernel(labels, vocab_table):
    mesh = plsc.VectorSubcoreMesh(core_axis_name="c", subcore_axis_name="s",
                                  num_cores=_NUM_CORES)
    call = pl.kernel(
        _lookup_body,
        out_type=jax.ShapeDtypeStruct((_B,), jnp.int32),
        mesh=mesh,
        scratch_types=[
            pltpu.VMEM((_B_PER_W,), jnp.int32),
            pltpu.VMEM((_V,), jnp.int32),
            pltpu.VMEM((_B_PER_W,), jnp.int32),
            pltpu.SemaphoreType.DMA,
        ],
        compiler_params=pltpu.CompilerParams(needs_layout_passes=False),
    )
    return call(labels, vocab_table)


# speedup vs baseline: 1.0200x; 1.0200x over previous
import jax
import jax.numpy as jnp
from jax import lax
from jax.experimental import pallas as pl
from jax.experimental.pallas import tpu as pltpu
from jax.experimental.pallas import tpu_sc as plsc

_V = 100
_B = 16384
_NUM_CORES = 1
_NUM_SUBCORES = 16
_NUM_WORKERS = _NUM_CORES * _NUM_SUBCORES
_B_PER_W = _B // _NUM_WORKERS
_LANES = 16


def _lookup_body(labels_hbm, vocab_hbm, out_hbm, labels_v, table_v, out_v,
                 sem0):
    wid = lax.axis_index("s") * _NUM_CORES + lax.axis_index("c")
    base = wid * _B_PER_W
    cp_labels = pltpu.async_copy(labels_hbm.at[pl.ds(base, _B_PER_W)],
                                 labels_v, sem0)
    cp_table = pltpu.async_copy(vocab_hbm, table_v, sem0)
    cp_labels.wait()
    cp_table.wait()
    half = _B_PER_W // 2

    out_cps = []
    for h, sem in ((0, sem0), (1, sem0)):
        @plsc.parallel_loop(h * half // _LANES, (h + 1) * half // _LANES,
                            unroll=4)
        def gather_chunk(i):
            idx = labels_v[pl.ds(i * _LANES, _LANES)]
            out_v[pl.ds(i * _LANES, _LANES)] = plsc.load_gather(table_v, [idx])
        out_cps.append(pltpu.async_copy(
            out_v.at[pl.ds(h * half, half)],
            out_hbm.at[pl.ds(base + h * half, half)], sem))
    for cp in out_cps:
        cp.wait()


@jax.jit
def kernel(labels, vocab_table):
    mesh = plsc.VectorSubcoreMesh(core_axis_name="c", subcore_axis_name="s",
                                  num_cores=_NUM_CORES)
    call = pl.kernel(
        _lookup_body,
        out_type=jax.ShapeDtypeStruct((_B,), jnp.int32),
        mesh=mesh,
        scratch_types=[
            pltpu.VMEM((_B_PER_W,), jnp.int32),
            pltpu.VMEM((_V,), jnp.int32),
            pltpu.VMEM((_B_PER_W,), jnp.int32),
            pltpu.SemaphoreType.DMA,
        ],
        compiler_params=pltpu.CompilerParams(needs_layout_passes=False),
    )
    return call(labels, vocab_table)


# speedup vs baseline: 1.0309x; 1.0108x over previous
import jax
import jax.numpy as jnp
from jax import lax
from jax.experimental import pallas as pl
from jax.experimental.pallas import tpu as pltpu
from jax.experimental.pallas import tpu_sc as plsc

_V = 100
_B = 16384
_NUM_CORES = 1
_NUM_SUBCORES = 16
_NUM_WORKERS = _NUM_CORES * _NUM_SUBCORES
_B_PER_W = _B // _NUM_WORKERS
_LANES = 16


def _lookup_body(labels_hbm, vocab_hbm, out_hbm, labels_v, table_v, out_v,
                 sem0):
    wid = lax.axis_index("s") * _NUM_CORES + lax.axis_index("c")
    base = wid * _B_PER_W
    cp_labels = pltpu.async_copy(labels_hbm.at[pl.ds(base, _B_PER_W)],
                                 labels_v, sem0)
    cp_table = pltpu.async_copy(vocab_hbm, table_v, sem0)
    cp_labels.wait()
    cp_table.wait()
    half = _B_PER_W // 2

    out_cps = []
    for h, sem in ((0, sem0), (1, sem0)):
        @plsc.parallel_loop(h * half // _LANES, (h + 1) * half // _LANES,
                            unroll=8)
        def gather_chunk(i):
            idx = labels_v[pl.ds(i * _LANES, _LANES)]
            out_v[pl.ds(i * _LANES, _LANES)] = plsc.load_gather(table_v, [idx])
        out_cps.append(pltpu.async_copy(
            out_v.at[pl.ds(h * half, half)],
            out_hbm.at[pl.ds(base + h * half, half)], sem))
    for cp in out_cps:
        cp.wait()


@jax.jit
def kernel(labels, vocab_table):
    mesh = plsc.VectorSubcoreMesh(core_axis_name="c", subcore_axis_name="s",
                                  num_cores=_NUM_CORES)
    call = pl.kernel(
        _lookup_body,
        out_type=jax.ShapeDtypeStruct((_B,), jnp.int32),
        mesh=mesh,
        scratch_types=[
            pltpu.VMEM((_B_PER_W,), jnp.int32),
            pltpu.VMEM((_V,), jnp.int32),
            pltpu.VMEM((_B_PER_W,), jnp.int32),
            pltpu.SemaphoreType.DMA,
        ],
        compiler_params=pltpu.CompilerParams(needs_layout_passes=False),
    )
    return call(labels, vocab_table)
